# baseline (speedup 1.0000x reference)
import functools

import jax
import jax.numpy as jnp
from jax import lax
from jax.experimental import pallas as pl
from jax.experimental.pallas import tpu as pltpu
from jax.experimental.pallas import tpu_sc as plsc

D = 64
PAD_D = 128
SEQ_LEN = 200
BLK = 128
TG = 8
N_CORES = 2
N_SUBCORES = 16
N_WORKERS = N_CORES * N_SUBCORES
TSTRIDE = 129


@functools.lru_cache(maxsize=None)
def _build(batch, vocab):
    n_tg = SEQ_LEN // TG
    nb = batch // BLK
    mesh = plsc.VectorSubcoreMesh(core_axis_name="c", subcore_axis_name="s")

    @functools.partial(
        pl.kernel,
        mesh=mesh,
        out_type=jax.ShapeDtypeStruct((SEQ_LEN, D // 8, nb, 8, BLK), jnp.float32),
        compiler_params=pltpu.CompilerParams(
            use_tc_tiling_on_sc=False, needs_layout_passes=False
        ),
        scratch_types=[
            pltpu.VMEM((TG, BLK), jnp.int32),
            pltpu.VMEM((TG, BLK), jnp.int32),
        ] + [pltpu.VMEM((BLK, D), jnp.float32) for _ in range(TG)] + [
            pltpu.VMEM((D // 8, 8, TSTRIDE), jnp.float32),
            pltpu.VMEM((D // 8, 8, TSTRIDE), jnp.float32),
            pltpu.VMEM((SEQ_LEN, PAD_D), jnp.float32),
            pltpu.SemaphoreType.DMA,
        ] + [pltpu.SemaphoreType.DMA for _ in range(TG)] + [
            pltpu.SemaphoreType.DMA,
            pltpu.SemaphoreType.DMA,
        ],
    )
    def emb(xT_hbm, tbl_hbm, pos_hbm, outT_hbm,
            idx_a, idx_b, r0, r1, r2, r3, r4, r5, r6, r7,
            trans_a, trans_b, posbuf,
            isem, g0, g1, g2, g3, g4, g5, g6, g7, osem_a, osem_b):
        wid = lax.axis_index("s") * N_CORES + lax.axis_index("c")
        b0 = pl.multiple_of(wid * BLK, BLK)
        idx_bufs = (idx_a, idx_b)
        rows_bufs = (r0, r1, r2, r3, r4, r5, r6, r7)
        gsems = (g0, g1, g2, g3, g4, g5, g6, g7)
        trans_bufs = (trans_a, trans_b)
        osems = (osem_a, osem_b)

        pltpu.sync_copy(pos_hbm, posbuf)

        iota = lax.iota(jnp.int32, 16)
        t_hi = [lax.div(iota + 16 * c, 8) for c in range(D // 16)]
        t_lo = [lax.rem(iota + 16 * c, 8) for c in range(D // 16)]

        def stage_idx(tg, buf):
            t0 = pl.multiple_of(tg * TG, TG)
            return pltpu.make_async_copy(
                xT_hbm.at[pl.ds(t0, TG), pl.ds(b0, BLK)], buf, isem
            )

        def start_gather(idx_buf, k, rows_buf, gsem):
            pltpu.async_copy(tbl_hbm.at[idx_buf.at[k]], rows_buf, gsem)

        def wait_gather(rows_buf, gsem):
            pltpu.make_async_copy(
                tbl_hbm.at[pl.ds(0, BLK)], rows_buf, gsem
            ).wait()

        def out_starts(t, trans_buf, osem):
            pltpu.async_copy(
                trans_buf.at[:, :, pl.ds(0, BLK)],
                outT_hbm.at[t, :, wid],
                osem,
            )

        def out_wait(t, trans_buf, osem):
            pltpu.make_async_copy(
                trans_buf.at[:, :, pl.ds(0, BLK)],
                outT_hbm.at[t, :, wid],
                osem,
            ).wait()

        def compute(t, rows_buf, trans_buf):
            posv = [posbuf[t, pl.ds(16 * c, 16)] for c in range(D // 16)]

            def tok_body(tok, carry):
                col = jnp.full((16,), tok, jnp.int32)
                for c in range(D // 16):
                    v = rows_buf[tok, pl.ds(16 * c, 16)] + posv[c]
                    plsc.store_scatter(trans_buf, [t_hi[c], t_lo[c], col], v)
                return carry

            lax.fori_loop(0, BLK, tok_body, 0, unroll=4)

        stage_idx(0, idx_a).start()
        stage_idx(0, idx_a).wait()
        for k in range(TG):
            start_gather(idx_a, k, rows_bufs[k], gsems[k])

        def tg_body(tg, carry):
            def one_tg(cur, nxt):
                @pl.when(tg + 1 < n_tg)
                def _():
                    stage_idx(tg + 1, nxt).start()

                for k in range(TG):
                    t = tg * TG + k
                    p = k & 1
                    wait_gather(rows_bufs[k], gsems[k])
                    if k == 0:
                        @pl.when(tg + 1 < n_tg)
                        def _():
                            stage_idx(tg + 1, nxt).wait()

                    @pl.when(t >= 2)
                    def _():
                        out_wait(t - 2, trans_bufs[p], osems[p])

                    compute(t, rows_bufs[k], trans_bufs[p])
                    out_starts(t, trans_bufs[p], osems[p])

                    @pl.when(tg + 1 < n_tg)
                    def _():
                        start_gather(nxt, k, rows_bufs[k], gsems[k])

            @pl.when(lax.rem(tg, 2) == 0)
            def _():
                one_tg(idx_a, idx_b)

            @pl.when(lax.rem(tg, 2) == 1)
            def _():
                one_tg(idx_b, idx_a)

            return carry

        lax.fori_loop(0, n_tg, tg_body, 0)
        out_wait(SEQ_LEN - 2, trans_a, osem_a)
        out_wait(SEQ_LEN - 1, trans_b, osem_b)

    return emb


def kernel(x, token_table, pos_table):
    b, t = x.shape
    vocab = token_table.shape[0]
    xT = (x.astype(jnp.int32) * 2).T
    tbl = jnp.pad(token_table, ((0, 0), (0, PAD_D - D))).reshape(2 * vocab, D)
    pos = jnp.pad(pos_table, ((0, 0), (0, PAD_D - D)))
    out5 = _build(b, vocab)(xT, tbl, pos)
    return out5.transpose(2, 4, 0, 1, 3).reshape(b, t, D)


# speedup vs baseline: 1.0201x; 1.0201x over previous
import functools

import jax
import jax.numpy as jnp
from jax import lax
from jax.experimental import pallas as pl
from jax.experimental.pallas import tpu as pltpu
from jax.experimental.pallas import tpu_sc as plsc

D = 64
PAD_D = 128
SEQ_LEN = 200
BLK = 128
TPC = 4
CHUNK = TPC * BLK
N_CORES = 2
N_SUBCORES = 16
N_WORKERS = N_CORES * N_SUBCORES
TSTRIDE = 129


@functools.lru_cache(maxsize=None)
def _build(batch, vocab):
    n_sc = SEQ_LEN // TPC
    nb = batch // BLK
    per_w = SEQ_LEN * BLK
    mesh = plsc.VectorSubcoreMesh(core_axis_name="c", subcore_axis_name="s")

    @functools.partial(
        pl.kernel,
        mesh=mesh,
        out_type=jax.ShapeDtypeStruct((SEQ_LEN, D // 8, nb, 8, BLK), jnp.float32),
        compiler_params=pltpu.CompilerParams(
            use_tc_tiling_on_sc=False, needs_layout_passes=False
        ),
        scratch_types=[
            pltpu.VMEM((CHUNK,), jnp.int32),
            pltpu.VMEM((CHUNK,), jnp.int32),
            pltpu.VMEM((CHUNK, D), jnp.float32),
            pltpu.VMEM((CHUNK, D), jnp.float32),
            pltpu.VMEM((D // 8, 8, TSTRIDE), jnp.float32),
            pltpu.VMEM((D // 8, 8, TSTRIDE), jnp.float32),
            pltpu.VMEM((SEQ_LEN, PAD_D), jnp.float32),
            pltpu.SemaphoreType.DMA,
            pltpu.SemaphoreType.DMA,
            pltpu.SemaphoreType.DMA,
            pltpu.SemaphoreType.DMA,
            pltpu.SemaphoreType.DMA,
        ],
    )
    def emb(xw_hbm, tbl_hbm, pos_hbm, outT_hbm,
            idx_a, idx_b, rows_a, rows_b, trans_a, trans_b, posbuf,
            isem, gsem_a, gsem_b, osem_a, osem_b):
        wid = lax.axis_index("s") * N_CORES + lax.axis_index("c")
        base = pl.multiple_of(wid * per_w, CHUNK)
        idx_bufs = (idx_a, idx_b)
        rows_bufs = (rows_a, rows_b)
        gsems = (gsem_a, gsem_b)
        trans_bufs = (trans_a, trans_b)
        osems = (osem_a, osem_b)

        pltpu.sync_copy(pos_hbm, posbuf)

        iota = lax.iota(jnp.int32, 16)
        t_hi = [lax.div(iota + 16 * c, 8) for c in range(D // 16)]
        t_lo = [lax.rem(iota + 16 * c, 8) for c in range(D // 16)]

        def stage_idx(sc, buf):
            off = pl.multiple_of(base + sc * CHUNK, CHUNK)
            return pltpu.make_async_copy(
                xw_hbm.at[pl.ds(off, CHUNK)], buf, isem
            )

        def start_gather(idx_buf, rows_buf, gsem):
            pltpu.async_copy(tbl_hbm.at[idx_buf], rows_buf, gsem)

        def wait_gather(rows_buf, gsem):
            pltpu.make_async_copy(
                tbl_hbm.at[pl.ds(0, CHUNK)], rows_buf, gsem
            ).wait()

        def out_copy(t, trans_buf, osem):
            return pltpu.make_async_copy(
                trans_buf.at[:, :, pl.ds(0, BLK)],
                outT_hbm.at[t, :, wid],
                osem,
            )

        def compute(sc, rows_buf):
            for r in range(TPC):
                t = sc * TPC + r
                tb = trans_bufs[r & 1]

                @pl.when(t >= 2)
                def _():
                    out_copy(t - 2, tb, osems[r & 1]).wait()

                posv = [posbuf[t, pl.ds(16 * c, 16)] for c in range(D // 16)]

                def tok_body(tok, carry):
                    col = jnp.full((16,), 0, jnp.int32) + tok
                    row = r * BLK + tok
                    for c in range(D // 16):
                        v = rows_buf[row, pl.ds(16 * c, 16)] + posv[c]
                        plsc.store_scatter(tb, [t_hi[c], t_lo[c], col], v)
                    return carry

                lax.fori_loop(0, BLK, tok_body, 0, unroll=4)
                out_copy(t, tb, osems[r & 1]).start()

        stage_idx(0, idx_a).start()
        stage_idx(0, idx_a).wait()
        start_gather(idx_a, rows_a, gsem_a)
        stage_idx(1, idx_b).start()

        def sc_body(sc, carry):
            def one(cur, nxt, p, q):
                wait_gather(rows_bufs[p], gsems[p])

                @pl.when(sc + 1 < n_sc)
                def _():
                    stage_idx(sc + 1, nxt).wait()
                    start_gather(nxt, rows_bufs[q], gsems[q])

                @pl.when(sc + 2 < n_sc)
                def _():
                    stage_idx(sc + 2, cur).start()

                compute(sc, rows_bufs[p])

            @pl.when(lax.rem(sc, 2) == 0)
            def _():
                one(idx_a, idx_b, 0, 1)

            @pl.when(lax.rem(sc, 2) == 1)
            def _():
                one(idx_b, idx_a, 1, 0)

            return carry

        lax.fori_loop(0, n_sc, sc_body, 0)
        out_copy(SEQ_LEN - 2, trans_a, osem_a).wait()
        out_copy(SEQ_LEN - 1, trans_b, osem_b).wait()

    return emb


def kernel(x, token_table, pos_table):
    b, t = x.shape
    vocab = token_table.shape[0]
    xw = (
        (x.astype(jnp.int32) * 2)
        .T.reshape(t, b // BLK, BLK)
        .transpose(1, 0, 2)
        .reshape(-1)
    )
    tbl = jnp.pad(token_table, ((0, 0), (0, PAD_D - D))).reshape(2 * vocab, D)
    pos = jnp.pad(pos_table, ((0, 0), (0, PAD_D - D)))
    out5 = _build(b, vocab)(xw, tbl, pos)
    return out5.transpose(2, 4, 0, 1, 3).reshape(b, t, D)
